# baseline (speedup 1.0000x reference)
import functools

import jax
import jax.numpy as jnp
from jax import lax
from jax.experimental import pallas as pl
from jax.experimental.pallas import tpu as pltpu
from jax.experimental.pallas import tpu_sc as plsc

_BATCH = 16384
_D = 32
_NW = 32
_BPW = _BATCH // _NW


def _gmf_body(uidx_hbm, iidx_hbm, utab_hbm, itab_hbm, out_hbm,
              uidx_v, iidx_v, urows_v, irows_v, sem_u, sem_i):
    wid = lax.axis_index("s") * 2 + lax.axis_index("c")
    base = wid * _BPW
    pltpu.sync_copy(uidx_hbm.at[pl.ds(base, _BPW)], uidx_v)
    pltpu.sync_copy(iidx_hbm.at[pl.ds(base, _BPW)], iidx_v)
    cu = pltpu.async_copy(utab_hbm.at[uidx_v], urows_v, sem_u)
    ci = pltpu.async_copy(itab_hbm.at[iidx_v], irows_v, sem_i)
    cu.wait()
    ci.wait()

    def mul_row(r, _):
        urows_v[r, pl.ds(0, 16)] = urows_v[r, pl.ds(0, 16)] * irows_v[r, pl.ds(0, 16)]
        urows_v[r, pl.ds(16, 16)] = urows_v[r, pl.ds(16, 16)] * irows_v[r, pl.ds(16, 16)]
        return _

    lax.fori_loop(0, _BPW, mul_row, None)
    pltpu.sync_copy(urows_v, out_hbm.at[pl.ds(base, _BPW)])


@jax.jit
def kernel(user_indices, item_indices, user_table, item_table):
    mesh = plsc.VectorSubcoreMesh(core_axis_name="c", subcore_axis_name="s")
    f = pl.kernel(
        _gmf_body,
        out_type=jax.ShapeDtypeStruct((_BATCH, _D), jnp.float32),
        mesh=mesh,
        scratch_types=[
            pltpu.VMEM((_BPW,), jnp.int32),
            pltpu.VMEM((_BPW,), jnp.int32),
            pltpu.VMEM((_BPW, _D), jnp.float32),
            pltpu.VMEM((_BPW, _D), jnp.float32),
            pltpu.SemaphoreType.DMA,
            pltpu.SemaphoreType.DMA,
        ],
        compiler_params=pltpu.CompilerParams(use_tc_tiling_on_sc=False),
    )
    return f(user_indices.astype(jnp.int32), item_indices.astype(jnp.int32),
             user_table, item_table)
